# Initial kernel scaffold; baseline (speedup 1.0000x reference)
import jax
import jax.numpy as jnp
from jax.experimental import pallas as pl


def kernel(x, table, b, c):
    raise NotImplementedError("write your pallas kernel here")


# speedup vs baseline: 1.2841x; 1.2841x over previous
import jax
import jax.numpy as jnp
from jax.experimental import pallas as pl
from jax.experimental.pallas import tpu as pltpu
from jax.experimental.pallas import tpu_sc as plsc

_W = 128


def kernel(x, table, b, c):
    n, k = x.shape
    num = n * k
    dim = table.shape[1]
    x_flat = x.reshape(1, num)

    mesh = plsc.VectorSubcoreMesh(core_axis_name="core",
                                  subcore_axis_name="subcore")

    @pl.kernel(
        out_type=(
            jax.ShapeDtypeStruct((num, dim), table.dtype),
            jax.ShapeDtypeStruct((1, num), b.dtype),
            jax.ShapeDtypeStruct((1, num), c.dtype),
        ),
        mesh=mesh,
        compiler_params=pltpu.CompilerParams(use_tc_tiling_on_sc=False),
    )
    def gather_kernel(x_hbm, table_hbm, b_hbm, c_hbm, y_hbm, bo_hbm, co_hbm):
        def body(i_vmem, y_vmem, bo_vmem, co_vmem):
            idx = i_vmem.at[0]
            pltpu.sync_copy(table_hbm.at[idx], y_vmem)
            pltpu.sync_copy(b_hbm.at[idx], bo_vmem.at[0])
            pltpu.sync_copy(c_hbm.at[idx], co_vmem.at[0])

        pltpu.emit_pipeline(
            body,
            grid=(num // _W,),
            in_specs=[pl.BlockSpec((1, _W), lambda i: (0, i))],
            out_specs=[
                pl.BlockSpec((_W, dim), lambda i: (i, 0)),
                pl.BlockSpec((1, _W), lambda i: (0, i)),
                pl.BlockSpec((1, _W), lambda i: (0, i)),
            ],
            core_axis_name=("core", "subcore"),
            dimension_semantics=(pltpu.PARALLEL,),
        )(x_hbm, y_hbm, bo_hbm, co_hbm)

    y, b_out, c_out = gather_kernel(x_flat, table, b, c)
    return (y.reshape(n, k, dim), b_out.reshape(n, k), c_out.reshape(n, k))


# speedup vs baseline: 1.5617x; 1.2162x over previous
import jax
import jax.numpy as jnp
from jax.experimental import pallas as pl
from jax.experimental.pallas import tpu as pltpu
from jax.experimental.pallas import tpu_sc as plsc

_W = 1024


def kernel(x, table, b, c):
    n, k = x.shape
    num = n * k
    dim = table.shape[1]
    x_flat = x.reshape(1, num)

    mesh = plsc.VectorSubcoreMesh(core_axis_name="core",
                                  subcore_axis_name="subcore")

    @pl.kernel(
        out_type=(
            jax.ShapeDtypeStruct((num, dim), table.dtype),
            jax.ShapeDtypeStruct((1, num), b.dtype),
            jax.ShapeDtypeStruct((1, num), c.dtype),
        ),
        mesh=mesh,
        scratch_types=[pltpu.SemaphoreType.DMA] * 3,
        compiler_params=pltpu.CompilerParams(use_tc_tiling_on_sc=False),
    )
    def gather_kernel(x_hbm, table_hbm, b_hbm, c_hbm, y_hbm, bo_hbm, co_hbm,
                      sem_y, sem_b, sem_c):
        def body(i_vmem, y_vmem, bo_vmem, co_vmem):
            idx = i_vmem.at[0]
            cp_y = pltpu.async_copy(table_hbm.at[idx], y_vmem, sem_y)
            cp_b = pltpu.async_copy(b_hbm.at[idx], bo_vmem.at[0], sem_b)
            cp_c = pltpu.async_copy(c_hbm.at[idx], co_vmem.at[0], sem_c)
            cp_y.wait()
            cp_b.wait()
            cp_c.wait()

        pltpu.emit_pipeline(
            body,
            grid=(num // _W,),
            in_specs=[pl.BlockSpec((1, _W), lambda i: (0, i))],
            out_specs=[
                pl.BlockSpec((_W, dim), lambda i: (i, 0)),
                pl.BlockSpec((1, _W), lambda i: (0, i)),
                pl.BlockSpec((1, _W), lambda i: (0, i)),
            ],
            core_axis_name=("core", "subcore"),
            dimension_semantics=(pltpu.PARALLEL,),
        )(x_hbm, y_hbm, bo_hbm, co_hbm)

    y, b_out, c_out = gather_kernel(x_flat, table, b, c)
    return (y.reshape(n, k, dim), b_out.reshape(n, k), c_out.reshape(n, k))


# speedup vs baseline: 1.5680x; 1.0040x over previous
import jax
import jax.numpy as jnp
from jax.experimental import pallas as pl
from jax.experimental.pallas import tpu as pltpu
from jax.experimental.pallas import tpu_sc as plsc

_W = 1664


def kernel(x, table, b, c):
    n, k = x.shape
    num = n * k
    dim = table.shape[1]
    x_flat = x.reshape(1, num)

    mesh = plsc.VectorSubcoreMesh(core_axis_name="core",
                                  subcore_axis_name="subcore")

    @pl.kernel(
        out_type=(
            jax.ShapeDtypeStruct((num, dim), table.dtype),
            jax.ShapeDtypeStruct((1, num), b.dtype),
            jax.ShapeDtypeStruct((1, num), c.dtype),
        ),
        mesh=mesh,
        scratch_types=[pltpu.SemaphoreType.DMA] * 3,
        compiler_params=pltpu.CompilerParams(use_tc_tiling_on_sc=False),
    )
    def gather_kernel(x_hbm, table_hbm, b_hbm, c_hbm, y_hbm, bo_hbm, co_hbm,
                      sem_y, sem_b, sem_c):
        def body(i_vmem, y_vmem, bo_vmem, co_vmem):
            idx = i_vmem.at[0]
            cp_y = pltpu.async_copy(table_hbm.at[idx], y_vmem, sem_y)
            cp_b = pltpu.async_copy(b_hbm.at[idx], bo_vmem.at[0], sem_b)
            cp_c = pltpu.async_copy(c_hbm.at[idx], co_vmem.at[0], sem_c)
            cp_y.wait()
            cp_b.wait()
            cp_c.wait()

        pltpu.emit_pipeline(
            body,
            grid=(num // _W,),
            in_specs=[pl.BlockSpec((1, _W), lambda i: (0, i))],
            out_specs=[
                pl.BlockSpec((_W, dim), lambda i: (i, 0)),
                pl.BlockSpec((1, _W), lambda i: (0, i)),
                pl.BlockSpec((1, _W), lambda i: (0, i)),
            ],
            core_axis_name=("core", "subcore"),
            dimension_semantics=(pltpu.PARALLEL,),
        )(x_hbm, y_hbm, bo_hbm, co_hbm)

    y, b_out, c_out = gather_kernel(x_flat, table, b, c)
    return (y.reshape(n, k, dim), b_out.reshape(n, k), c_out.reshape(n, k))
